# baseline (speedup 1.0000x reference)
import jax
import jax.numpy as jnp
from jax import lax
from jax.experimental import pallas as pl
from jax.experimental.pallas import tpu as pltpu

_B, _C, _H, _W = 8, 21, 512, 512
_HW = _H * _W
_EPS = 1e-05
_K = 256
_NJ = (_HW // 128) // _K


def _p2_body(pred_ref, out_ref):
    p = pred_ref[0].reshape(_HW // 1024, 8, 128)
    out_ref[0] = jnp.sum(p * p, axis=0)


def kernel(prediction, target):
    pred3 = prediction.reshape(_B * _C, _HW // 128, 128)

    p2 = pl.pallas_call(
        _p2_body,
        grid=(_B * _C,),
        in_specs=[pl.BlockSpec((1, _HW // 128, 128), lambda r: (r, 0, 0))],
        out_specs=pl.BlockSpec((1, 8, 128), lambda r: (r, 0, 0)),
        out_shape=jax.ShapeDtypeStruct((_B * _C, 8, 128), jnp.float32),
        compiler_params=pltpu.CompilerParams(
            dimension_semantics=("arbitrary",)),
    )(pred3).sum(axis=(1, 2)).reshape(_B, _C)

    pred3 = prediction.reshape(_B, _C, _HW)
    tgt = target.astype(jnp.int32).reshape(_B, 1, _HW)
    oh = (tgt == lax.broadcasted_iota(jnp.int32, (1, _C, 1), 1))
    inter = jnp.sum(jnp.where(oh, pred3, 0.0), axis=2)
    cnt = jnp.sum(oh.astype(jnp.float32), axis=2) * jnp.ones((_B, _C))
    dice = (2.0 * inter + _EPS) / (p2 + cnt + _EPS)
    return 1.0 - dice.mean()


# speedup vs baseline: 1.3546x; 1.3546x over previous
import jax
import jax.numpy as jnp
from jax import lax
from jax.experimental import pallas as pl
from jax.experimental.pallas import tpu as pltpu
from jax.experimental.pallas import tpu_sc as plsc

_B, _C, _H, _W = 8, 21, 512, 512
_HW = _H * _W
_EPS = 1e-05

_K = 256
_NJ = (_HW // 128) // _K


def _tc_body(pred_ref, tgt_ref, out_ref):
    j = pl.program_id(1)
    p = pred_ref[0]
    t = tgt_ref[0]
    cls = lax.broadcasted_iota(jnp.int32, (_C, 1, 1), 0)
    m = cls == t
    inter = jnp.sum(jnp.where(m, p, 0.0), axis=1)
    p2 = jnp.sum(p * p, axis=1)
    part = jnp.concatenate([inter, p2], axis=0)

    @pl.when(j == 0)
    def _():
        out_ref[0] = part

    @pl.when(j != 0)
    def _():
        out_ref[0] += part


_NC, _NS = 2, 16
_NW = _NC * _NS
_PXW = (_B * _HW) // _NW
_CHUNK_SC = 1024
_NCH = _PXW // _CHUNK_SC


def _sc_hist_body(tgt_hbm, out_hbm, tbuf, bins):
    wid = lax.axis_index("s") * _NC + lax.axis_index("c")
    base = wid * _PXW
    ones = jnp.ones((16,), jnp.float32)
    zeros = jnp.zeros((16,), jnp.float32)
    li = lax.iota(jnp.int32, 16)

    def zero_row(r, carry):
        bins[pl.ds(r * 16, 16)] = zeros
        return carry

    lax.fori_loop(0, _C * _C, zero_row, 0)

    def chunk(i, carry):
        pltpu.sync_copy(tgt_hbm.at[pl.ds(base + i * _CHUNK_SC, _CHUNK_SC)],
                        tbuf)
        for s in range(_CHUNK_SC // 32):
            ta = tbuf[pl.ds(s * 32, 16)]
            tb = tbuf[pl.ds(s * 32 + 16, 16)]
            plsc.addupdate_scatter(bins, [(ta * _C + tb) * 16 + li], ones)
        return carry

    lax.fori_loop(0, _NCH, chunk, 0)
    pltpu.sync_copy(bins, out_hbm.at[wid])


def _sc_hist(tgt_flat):
    mesh = plsc.VectorSubcoreMesh(core_axis_name="c", subcore_axis_name="s")
    return pl.kernel(
        _sc_hist_body,
        out_type=jax.ShapeDtypeStruct((_NW, _C * _C * 16), jnp.float32),
        mesh=mesh,
        scratch_types=[
            pltpu.VMEM((_CHUNK_SC,), jnp.int32),
            pltpu.VMEM((_C * _C * 16,), jnp.float32),
        ],
        compiler_params=pltpu.CompilerParams(needs_layout_passes=False),
    )(tgt_flat)


def kernel(prediction, target):
    tgt = target.astype(jnp.int32)
    pred4 = prediction.reshape(_B, _C, _HW // 128, 128)
    tgt4 = tgt.reshape(_B, 1, _HW // 128, 128)

    sums = pl.pallas_call(
        _tc_body,
        grid=(_B, _NJ),
        in_specs=[
            pl.BlockSpec((1, _C, _K, 128), lambda b, j: (b, 0, j, 0)),
            pl.BlockSpec((1, 1, _K, 128), lambda b, j: (b, 0, j, 0)),
        ],
        out_specs=pl.BlockSpec((1, 2 * _C, 128), lambda b, j: (b, 0, 0)),
        out_shape=jax.ShapeDtypeStruct((_B, 2 * _C, 128), jnp.float32),
        compiler_params=pltpu.CompilerParams(
            dimension_semantics=("parallel", "arbitrary")),
    )(pred4, tgt4)

    hist = _sc_hist(tgt.reshape(_B * _HW))

    s = sums.sum(axis=-1)
    inter = s[:, :_C]
    p2 = s[:, _C:]
    joint = hist.reshape(_B, _NW // _B, _C, _C, 16).sum(axis=(1, 4))
    cnt = joint.sum(axis=2) + joint.sum(axis=1)
    dice = (2.0 * inter + _EPS) / (p2 + cnt + _EPS)
    return 1.0 - dice.mean()


# speedup vs baseline: 1.4198x; 1.0481x over previous
import jax
import jax.numpy as jnp
from jax import lax
from jax.experimental import pallas as pl
from jax.experimental.pallas import tpu as pltpu
from jax.experimental.pallas import tpu_sc as plsc

_B, _C, _H, _W = 8, 21, 512, 512
_HW = _H * _W
_EPS = 1e-05

_K = 512
_NJ = (_HW // 128) // _K


def _tc_body(pred_ref, tgt_ref, out_ref):
    j = pl.program_id(1)
    p = pred_ref[0]
    t = tgt_ref[0]
    cls = lax.broadcasted_iota(jnp.int32, (_C, 1, 1), 0)
    m = cls == t
    inter = jnp.sum(jnp.where(m, p, 0.0), axis=1)
    p2 = jnp.sum(p * p, axis=1)
    part = jnp.concatenate([inter, p2], axis=0)

    @pl.when(j == 0)
    def _():
        out_ref[0] = part

    @pl.when(j != 0)
    def _():
        out_ref[0] += part


_NC, _NS = 2, 16
_NW = _NC * _NS
_PXW = (_B * _HW) // _NW
_CHUNK_SC = 1024
_NCH = _PXW // _CHUNK_SC


def _sc_hist_body(tgt_hbm, out_hbm, tbuf, bins):
    wid = lax.axis_index("s") * _NC + lax.axis_index("c")
    base = wid * _PXW
    ones = jnp.ones((16,), jnp.float32)
    zeros = jnp.zeros((16,), jnp.float32)
    li = lax.iota(jnp.int32, 16)

    def zero_row(r, carry):
        bins[pl.ds(r * 16, 16)] = zeros
        return carry

    lax.fori_loop(0, _C * _C, zero_row, 0)

    def chunk(i, carry):
        pltpu.sync_copy(tgt_hbm.at[pl.ds(base + i * _CHUNK_SC, _CHUNK_SC)],
                        tbuf)
        for s in range(_CHUNK_SC // 32):
            ta = tbuf[pl.ds(s * 32, 16)]
            tb = tbuf[pl.ds(s * 32 + 16, 16)]
            plsc.addupdate_scatter(bins, [(ta * _C + tb) * 16 + li], ones)
        return carry

    lax.fori_loop(0, _NCH, chunk, 0)
    pltpu.sync_copy(bins, out_hbm.at[wid])


def _sc_hist(tgt_flat):
    mesh = plsc.VectorSubcoreMesh(core_axis_name="c", subcore_axis_name="s")
    return pl.kernel(
        _sc_hist_body,
        out_type=jax.ShapeDtypeStruct((_NW, _C * _C * 16), jnp.float32),
        mesh=mesh,
        scratch_types=[
            pltpu.VMEM((_CHUNK_SC,), jnp.int32),
            pltpu.VMEM((_C * _C * 16,), jnp.float32),
        ],
        compiler_params=pltpu.CompilerParams(needs_layout_passes=False),
    )(tgt_flat)


def kernel(prediction, target):
    tgt = target.astype(jnp.int32)
    pred4 = prediction.reshape(_B, _C, _HW // 128, 128)
    tgt4 = tgt.reshape(_B, 1, _HW // 128, 128)

    sums = pl.pallas_call(
        _tc_body,
        grid=(_B, _NJ),
        in_specs=[
            pl.BlockSpec((1, _C, _K, 128), lambda b, j: (b, 0, j, 0)),
            pl.BlockSpec((1, 1, _K, 128), lambda b, j: (b, 0, j, 0)),
        ],
        out_specs=pl.BlockSpec((1, 2 * _C, 128), lambda b, j: (b, 0, 0)),
        out_shape=jax.ShapeDtypeStruct((_B, 2 * _C, 128), jnp.float32),
        compiler_params=pltpu.CompilerParams(
            dimension_semantics=("parallel", "arbitrary")),
    )(pred4, tgt4)

    hist = _sc_hist(tgt.reshape(_B * _HW))

    s = sums.sum(axis=-1)
    inter = s[:, :_C]
    p2 = s[:, _C:]
    joint = hist.reshape(_B, _NW // _B, _C, _C, 16).sum(axis=(1, 4))
    cnt = joint.sum(axis=2) + joint.sum(axis=1)
    dice = (2.0 * inter + _EPS) / (p2 + cnt + _EPS)
    return 1.0 - dice.mean()


# speedup vs baseline: 1.4500x; 1.0213x over previous
import jax
import jax.numpy as jnp
from jax import lax
from jax.experimental import pallas as pl
from jax.experimental.pallas import tpu as pltpu
from jax.experimental.pallas import tpu_sc as plsc

_B, _C, _H, _W = 8, 21, 512, 512
_HW = _H * _W
_EPS = 1e-05

_K = 1024
_NJ = (_HW // 128) // _K


def _tc_body(pred_ref, tgt_ref, out_ref):
    j = pl.program_id(1)
    p = pred_ref[0]
    t = tgt_ref[0]
    cls = lax.broadcasted_iota(jnp.int32, (_C, 1, 1), 0)
    m = cls == t
    inter = jnp.sum(jnp.where(m, p, 0.0), axis=1)
    p2 = jnp.sum(p * p, axis=1)
    part = jnp.concatenate([inter, p2], axis=0)

    @pl.when(j == 0)
    def _():
        out_ref[0] = part

    @pl.when(j != 0)
    def _():
        out_ref[0] += part


_NC, _NS = 2, 16
_NW = _NC * _NS
_PXW = (_B * _HW) // _NW
_CHUNK_SC = 1024
_NCH = _PXW // _CHUNK_SC


def _sc_hist_body(tgt_hbm, out_hbm, tbuf, bins):
    wid = lax.axis_index("s") * _NC + lax.axis_index("c")
    base = wid * _PXW
    ones = jnp.ones((16,), jnp.float32)
    zeros = jnp.zeros((16,), jnp.float32)
    li = lax.iota(jnp.int32, 16)

    def zero_row(r, carry):
        bins[pl.ds(r * 16, 16)] = zeros
        return carry

    lax.fori_loop(0, _C * _C, zero_row, 0)

    def chunk(i, carry):
        pltpu.sync_copy(tgt_hbm.at[pl.ds(base + i * _CHUNK_SC, _CHUNK_SC)],
                        tbuf)
        for s in range(_CHUNK_SC // 32):
            ta = tbuf[pl.ds(s * 32, 16)]
            tb = tbuf[pl.ds(s * 32 + 16, 16)]
            plsc.addupdate_scatter(bins, [(ta * _C + tb) * 16 + li], ones)
        return carry

    lax.fori_loop(0, _NCH, chunk, 0)
    pltpu.sync_copy(bins, out_hbm.at[wid])


def _sc_hist(tgt_flat):
    mesh = plsc.VectorSubcoreMesh(core_axis_name="c", subcore_axis_name="s")
    return pl.kernel(
        _sc_hist_body,
        out_type=jax.ShapeDtypeStruct((_NW, _C * _C * 16), jnp.float32),
        mesh=mesh,
        scratch_types=[
            pltpu.VMEM((_CHUNK_SC,), jnp.int32),
            pltpu.VMEM((_C * _C * 16,), jnp.float32),
        ],
        compiler_params=pltpu.CompilerParams(needs_layout_passes=False),
    )(tgt_flat)


def kernel(prediction, target):
    tgt = target.astype(jnp.int32)
    pred4 = prediction.reshape(_B, _C, _HW // 128, 128)
    tgt4 = tgt.reshape(_B, 1, _HW // 128, 128)

    sums = pl.pallas_call(
        _tc_body,
        grid=(_B, _NJ),
        in_specs=[
            pl.BlockSpec((1, _C, _K, 128), lambda b, j: (b, 0, j, 0)),
            pl.BlockSpec((1, 1, _K, 128), lambda b, j: (b, 0, j, 0)),
        ],
        out_specs=pl.BlockSpec((1, 2 * _C, 128), lambda b, j: (b, 0, 0)),
        out_shape=jax.ShapeDtypeStruct((_B, 2 * _C, 128), jnp.float32),
        compiler_params=pltpu.CompilerParams(
            dimension_semantics=("parallel", "arbitrary")),
    )(pred4, tgt4)

    hist = _sc_hist(tgt.reshape(_B * _HW))

    s = sums.sum(axis=-1)
    inter = s[:, :_C]
    p2 = s[:, _C:]
    joint = hist.reshape(_B, _NW // _B, _C, _C, 16).sum(axis=(1, 4))
    cnt = joint.sum(axis=2) + joint.sum(axis=1)
    dice = (2.0 * inter + _EPS) / (p2 + cnt + _EPS)
    return 1.0 - dice.mean()
